# Initial kernel scaffold; baseline (speedup 1.0000x reference)
import jax
import jax.numpy as jnp
from jax.experimental import pallas as pl


def kernel(x, edge_index, W1, b1, W2, b2, c1, c2, idx1, idx2):
    raise NotImplementedError("write your pallas kernel here")


# speedup vs baseline: 19.3545x; 19.3545x over previous
import jax
import jax.numpy as jnp
import numpy as np
from jax import lax
from jax.experimental import pallas as pl
from jax.experimental.pallas import tpu as pltpu
from jax.experimental.pallas import tpu_sc as plsc

N = 10000
D = 128
E = 320000
NSPEC = 1000
NSPEC_P = 1024
ALPHA = 1.0

NC, NS = 2, 16
NW = NC * NS
BATCH = 128
KB = 80
EP = NW * KB * BATCH
NPAD = N + 112
RPW = NPAD // NS
ZR = RPW // 2

BROW = 2000
GRID = N // BROW

_f32 = jnp.float32
_HIGH = lax.Precision.HIGHEST


_MESH = plsc.VectorSubcoreMesh(core_axis_name="c", subcore_axis_name="s")


def _deg_body(dst_hbm, out_hbm, dst_v, ones_v, hist_sh):
    cid = lax.axis_index("c")
    sid = lax.axis_index("s")
    wid = cid * NS + sid
    base = sid * RPW

    @pl.loop(0, BATCH)
    def _(i):
        for k in range(D // 16):
            ones_v[i, pl.ds(k * 16, 16)] = jnp.zeros((16,), _f32)

    for k in range(RPW // BATCH):
        pltpu.sync_copy(ones_v, hist_sh.at[pl.ds(base + k * BATCH, BATCH)])
    rem = RPW % BATCH
    if rem:
        pltpu.sync_copy(ones_v.at[pl.ds(0, rem)],
                        hist_sh.at[pl.ds(base + (RPW // BATCH) * BATCH, rem)])

    @pl.loop(0, BATCH)
    def _(i):
        for k in range(D // 16):
            ones_v[i, pl.ds(k * 16, 16)] = jnp.ones((16,), _f32)

    pltpu.sync_copy(dst_hbm.at[wid], dst_v)
    plsc.subcore_barrier()

    @pl.loop(0, KB)
    def _(b):
        pltpu.sync_copy(ones_v, hist_sh.at[dst_v.at[b]], add=True)

    plsc.subcore_barrier()
    pltpu.sync_copy(hist_sh.at[pl.ds(base, RPW)],
                    out_hbm.at[pl.ds(cid * NPAD + base, RPW)])


def _deg_call(dst3):
    f = pl.kernel(
        _deg_body,
        out_type=jax.ShapeDtypeStruct((NC * NPAD, D), _f32),
        mesh=_MESH,
        scratch_types=[
            pltpu.VMEM((KB, BATCH), jnp.int32),
            pltpu.VMEM((BATCH, D), _f32),
            pltpu.VMEM_SHARED((NPAD, D), _f32),
        ],
    )
    return f(dst3)


CH = 16
NCHUNK = KB // CH


def _msg_body(hw_hbm, src_hbm, dst_hbm, out_hbm,
              src_v, dst_v, rows0, rows1, acc_sh, sem0, sem1):
    cid = lax.axis_index("c")
    sid = lax.axis_index("s")
    wid = cid * NS + sid
    base = sid * RPW

    @pl.loop(0, BATCH)
    def _(i):
        for k in range(D // 16):
            rows0[i, pl.ds(k * 16, 16)] = jnp.zeros((16,), _f32)

    for k in range(RPW // BATCH):
        pltpu.sync_copy(rows0, acc_sh.at[pl.ds(base + k * BATCH, BATCH)])
    rem = RPW % BATCH
    if rem:
        pltpu.sync_copy(rows0.at[pl.ds(0, rem)],
                        acc_sh.at[pl.ds(base + (RPW // BATCH) * BATCH, rem)])
    plsc.subcore_barrier()

    @pl.loop(0, NCHUNK)
    def _(c):
        pltpu.sync_copy(src_hbm.at[wid, pl.ds(c * CH, CH)], src_v)
        pltpu.sync_copy(dst_hbm.at[wid, pl.ds(c * CH, CH)], dst_v)

        @pl.loop(0, CH)
        def _(b):
            pltpu.async_copy(hw_hbm.at[src_v.at[b]], rows1, sem1).wait()
            pltpu.sync_copy(rows1, acc_sh.at[dst_v.at[b]], add=True)

    plsc.subcore_barrier()
    pltpu.sync_copy(acc_sh.at[pl.ds(base, RPW)],
                    out_hbm.at[pl.ds(cid * NPAD + base, RPW)])


def _msg_call(hw, src3, dst3):
    f = pl.kernel(
        _msg_body,
        out_type=jax.ShapeDtypeStruct((NC * NPAD, D), _f32),
        mesh=_MESH,
        scratch_types=[
            pltpu.VMEM((CH, BATCH), jnp.int32),
            pltpu.VMEM((CH, BATCH), jnp.int32),
            pltpu.VMEM((BATCH, D), _f32),
            pltpu.VMEM((BATCH, D), _f32),
            pltpu.VMEM_SHARED((NPAD, D), _f32),
            pltpu.SemaphoreType.DMA,
            pltpu.SemaphoreType.DMA,
        ],
    )
    return f(hw, src3, dst3)


def _dw_from_coeffs(cpad, ipad):
    r = ipad[:NSPEC_P]
    s = ipad[NSPEC_P:]
    j_a = lax.broadcasted_iota(jnp.int32, (D, NSPEC_P), 0)
    j_b = lax.broadcasted_iota(jnp.int32, (NSPEC_P, D), 1)
    scale = _f32(2.0 * np.pi / D)
    ang_a = ((j_a * r[None, :]) % D).astype(_f32) * scale
    ang_b = ((s[:, None] * j_b) % D).astype(_f32) * scale
    ca = jnp.cos(ang_a) * cpad[None, :]
    sa = jnp.sin(ang_a) * cpad[None, :]
    cb = jnp.cos(ang_b)
    sb = jnp.sin(ang_b)
    dw = (jnp.dot(ca, cb, preferred_element_type=_f32, precision=_HIGH)
          - jnp.dot(sa, sb, preferred_element_type=_f32, precision=_HIGH))
    return dw * _f32(ALPHA / (D * D))


def _fourier_body(x_ref, w1_ref, c1_ref, i1_ref, c2_ref, i2_ref,
                  hw1_ref, embf_ref, dw1_s, dw2_s):
    @pl.when(pl.program_id(0) == 0)
    def _():
        dw1_s[...] = _dw_from_coeffs(c1_ref[0], i1_ref[0])
        dw2_s[...] = _dw_from_coeffs(c2_ref[0], i2_ref[0])

    xb = x_ref[...]
    hw1_ref[...] = jnp.dot(xb, w1_ref[...],
                           preferred_element_type=_f32, precision=_HIGH)
    xf = jnp.maximum(
        jnp.dot(xb, dw1_s[...], preferred_element_type=_f32, precision=_HIGH),
        0.0)
    embf_ref[...] = jnp.dot(xf, dw2_s[...],
                            preferred_element_type=_f32, precision=_HIGH)


def _fourier_call(x, w1, cp1, ip1, cp2, ip2):
    full = lambda shape: pl.BlockSpec(shape, lambda i: (0, 0))
    return pl.pallas_call(
        _fourier_body,
        grid=(GRID,),
        in_specs=[
            pl.BlockSpec((BROW, D), lambda i: (i, 0)),
            full((D, D)),
            full((1, NSPEC_P)),
            full((1, 2 * NSPEC_P)),
            full((1, NSPEC_P)),
            full((1, 2 * NSPEC_P)),
        ],
        out_specs=[
            pl.BlockSpec((BROW, D), lambda i: (i, 0)),
            pl.BlockSpec((BROW, D), lambda i: (i, 0)),
        ],
        out_shape=[
            jax.ShapeDtypeStruct((N, D), _f32),
            jax.ShapeDtypeStruct((N, D), _f32),
        ],
        scratch_shapes=[
            pltpu.VMEM((D, D), _f32),
            pltpu.VMEM((D, D), _f32),
        ],
    )(x, w1, cp1, ip1, cp2, ip2)


def _prep_body(h0_ref, h1_ref, hw1_ref, dinv_ref, hw1s_ref):
    deg = h0_ref[:, 0:1] + h1_ref[:, 0:1] + 1.0
    dinv = lax.rsqrt(deg)
    dinv_b = jnp.broadcast_to(dinv, hw1_ref.shape)
    dinv_ref[...] = dinv_b
    hw1s_ref[...] = hw1_ref[...] * dinv_b


def _prep_call(h0, h1, hw1):
    return pl.pallas_call(
        _prep_body,
        grid=(GRID,),
        in_specs=[
            pl.BlockSpec((BROW, D), lambda i: (i, 0)),
            pl.BlockSpec((BROW, D), lambda i: (i, 0)),
            pl.BlockSpec((BROW, D), lambda i: (i, 0)),
        ],
        out_specs=[
            pl.BlockSpec((BROW, D), lambda i: (i, 0)),
            pl.BlockSpec((BROW, D), lambda i: (i, 0)),
        ],
        out_shape=[
            jax.ShapeDtypeStruct((N, D), _f32),
            jax.ShapeDtypeStruct((N, D), _f32),
        ],
    )(h0, h1, hw1)


def _mid_body(a0_ref, a1_ref, hw1s_ref, dinv_ref, b1_ref, w2_ref, hw2s_ref):
    h1 = jnp.maximum(
        dinv_ref[...] * (a0_ref[...] + a1_ref[...] + hw1s_ref[...])
        + b1_ref[...], 0.0)
    hw2s_ref[...] = dinv_ref[...] * jnp.dot(
        h1, w2_ref[...], preferred_element_type=_f32, precision=_HIGH)


def _mid_call(a0, a1, hw1s, dinv, b1r, w2):
    row = pl.BlockSpec((BROW, D), lambda i: (i, 0))
    return pl.pallas_call(
        _mid_body,
        grid=(GRID,),
        in_specs=[row, row, row, row,
                  pl.BlockSpec((1, D), lambda i: (0, 0)),
                  pl.BlockSpec((D, D), lambda i: (0, 0))],
        out_specs=row,
        out_shape=jax.ShapeDtypeStruct((N, D), _f32),
    )(a0, a1, hw1s, dinv, b1r, w2)


def _final_body(a0_ref, a1_ref, hw2s_ref, dinv_ref, b2_ref, embf_ref,
                base_ref, total_ref):
    base = (dinv_ref[...] * (a0_ref[...] + a1_ref[...] + hw2s_ref[...])
            + b2_ref[...])
    base_ref[...] = base
    total_ref[...] = base + embf_ref[...]


def _final_call(a0, a1, hw2s, dinv, b2r, embf):
    row = pl.BlockSpec((BROW, D), lambda i: (i, 0))
    return pl.pallas_call(
        _final_body,
        grid=(GRID,),
        in_specs=[row, row, row, row,
                  pl.BlockSpec((1, D), lambda i: (0, 0)),
                  row],
        out_specs=[row, row],
        out_shape=[
            jax.ShapeDtypeStruct((N, D), _f32),
            jax.ShapeDtypeStruct((N, D), _f32),
        ],
    )(a0, a1, hw2s, dinv, b2r, embf)


def kernel(x, edge_index, W1, b1, W2, b2, c1, c2, idx1, idx2):
    src = edge_index[0]
    dst = edge_index[1]
    npad = EP - E
    pad_ids = jnp.arange(npad, dtype=jnp.int32)
    psrc = jnp.concatenate([src, pad_ids % 997])
    pdst = jnp.concatenate([dst, N + (pad_ids % 112)])
    src3 = psrc.reshape(NW, KB, BATCH)
    dst3 = pdst.reshape(NW, KB, BATCH)

    def pack(c, idx):
        cp = jnp.zeros((1, NSPEC_P), _f32).at[0, :NSPEC].set(c)
        ip = jnp.zeros((1, 2 * NSPEC_P), jnp.int32)
        ip = ip.at[0, :NSPEC].set(idx[0]).at[0, NSPEC_P:NSPEC_P + NSPEC].set(idx[1])
        return cp, ip

    cp1, ip1 = pack(c1, idx1)
    cp2, ip2 = pack(c2, idx2)

    hist = _deg_call(dst3)
    hw1, embf = _fourier_call(x, W1, cp1, ip1, cp2, ip2)

    h0 = hist[:N]
    h1 = hist[NPAD:NPAD + N]
    dinv, hw1s = _prep_call(h0, h1, hw1)

    def msg(hw):
        acc = _msg_call(hw, src3, dst3)
        return acc[:N], acc[NPAD:NPAD + N]

    a10, a11 = msg(hw1s)
    hw2s = _mid_call(a10, a11, hw1s, dinv, b1.reshape(1, D), W2)

    a20, a21 = msg(hw2s)
    emb_base, emb_total = _final_call(a20, a21, hw2s, dinv,
                                      b2.reshape(1, D), embf)
    return (emb_total, emb_base, embf)


# speedup vs baseline: 25.9902x; 1.3428x over previous
import jax
import jax.numpy as jnp
import numpy as np
from jax import lax
from jax.experimental import pallas as pl
from jax.experimental.pallas import tpu as pltpu
from jax.experimental.pallas import tpu_sc as plsc

N = 10000
D = 128
E = 320000
NSPEC = 1000
NSPEC_P = 1024
ALPHA = 1.0

NC, NS = 2, 16
NW = NC * NS
BATCH = 128
KB = 80
EP = NW * KB * BATCH
NPAD = N + 112
RPW = NPAD // NS
ZR = RPW // 2

BROW = 2000
GRID = N // BROW

_f32 = jnp.float32
_HIGH = lax.Precision.HIGHEST


_MESH = plsc.VectorSubcoreMesh(core_axis_name="c", subcore_axis_name="s")


def _deg_body(dst_hbm, out_hbm, dst_v, ones_v, hist_sh):
    cid = lax.axis_index("c")
    sid = lax.axis_index("s")
    wid = cid * NS + sid
    base = sid * RPW

    @pl.loop(0, BATCH)
    def _(i):
        for k in range(D // 16):
            ones_v[i, pl.ds(k * 16, 16)] = jnp.zeros((16,), _f32)

    for k in range(RPW // BATCH):
        pltpu.sync_copy(ones_v, hist_sh.at[pl.ds(base + k * BATCH, BATCH)])
    rem = RPW % BATCH
    if rem:
        pltpu.sync_copy(ones_v.at[pl.ds(0, rem)],
                        hist_sh.at[pl.ds(base + (RPW // BATCH) * BATCH, rem)])

    @pl.loop(0, BATCH)
    def _(i):
        for k in range(D // 16):
            ones_v[i, pl.ds(k * 16, 16)] = jnp.ones((16,), _f32)

    pltpu.sync_copy(dst_hbm.at[wid], dst_v)
    plsc.subcore_barrier()

    @pl.loop(0, KB)
    def _(b):
        pltpu.sync_copy(ones_v, hist_sh.at[dst_v.at[b]], add=True)

    plsc.subcore_barrier()
    pltpu.sync_copy(hist_sh.at[pl.ds(base, RPW)],
                    out_hbm.at[pl.ds(cid * NPAD + base, RPW)])


def _deg_call(dst3):
    f = pl.kernel(
        _deg_body,
        out_type=jax.ShapeDtypeStruct((NC * NPAD, D), _f32),
        mesh=_MESH,
        scratch_types=[
            pltpu.VMEM((KB, BATCH), jnp.int32),
            pltpu.VMEM((BATCH, D), _f32),
            pltpu.VMEM_SHARED((NPAD, D), _f32),
        ],
    )
    return f(dst3)


CH = 16
NCHUNK = KB // CH


def _msg_body(hw_hbm, src_hbm, dst_hbm, out_hbm,
              src_a, dst_a, src_b, dst_b, rows0, rows1, acc_sh,
              sem0, sem1, semi):
    cid = lax.axis_index("c")
    sid = lax.axis_index("s")
    wid = cid * NS + sid
    base = sid * RPW

    @pl.loop(0, BATCH)
    def _(i):
        for k in range(D // 16):
            rows0[i, pl.ds(k * 16, 16)] = jnp.zeros((16,), _f32)

    for k in range(RPW // BATCH):
        pltpu.sync_copy(rows0, acc_sh.at[pl.ds(base + k * BATCH, BATCH)])
    rem = RPW % BATCH
    if rem:
        pltpu.sync_copy(rows0.at[pl.ds(0, rem)],
                        acc_sh.at[pl.ds(base + (RPW // BATCH) * BATCH, rem)])
    plsc.subcore_barrier()

    pltpu.sync_copy(src_hbm.at[wid, pl.ds(0, CH)], src_a)
    pltpu.sync_copy(dst_hbm.at[wid, pl.ds(0, CH)], dst_a)
    for c in range(NCHUNK):
        cur_s, cur_d = (src_a, dst_a) if c % 2 == 0 else (src_b, dst_b)
        nxt_s, nxt_d = (src_b, dst_b) if c % 2 == 0 else (src_a, dst_a)
        if c + 1 < NCHUNK:
            pltpu.async_copy(src_hbm.at[wid, pl.ds((c + 1) * CH, CH)], nxt_s, semi)
            pltpu.async_copy(dst_hbm.at[wid, pl.ds((c + 1) * CH, CH)], nxt_d, semi)
        pltpu.async_copy(hw_hbm.at[cur_s.at[0]], rows0, sem0)
        pltpu.async_copy(hw_hbm.at[cur_s.at[1]], rows1, sem1)

        @pl.loop(0, CH // 2)
        def _(p, cur_s=cur_s, cur_d=cur_d):
            b0 = p * 2
            b1 = b0 + 1
            pltpu.make_async_copy(hw_hbm.at[cur_s.at[b0]], rows0, sem0).wait()
            pltpu.sync_copy(rows0, acc_sh.at[cur_d.at[b0]], add=True)

            @pl.when(p < CH // 2 - 1)
            def _():
                pltpu.async_copy(hw_hbm.at[cur_s.at[b0 + 2]], rows0, sem0)

            pltpu.make_async_copy(hw_hbm.at[cur_s.at[b1]], rows1, sem1).wait()
            pltpu.sync_copy(rows1, acc_sh.at[cur_d.at[b1]], add=True)

            @pl.when(p < CH // 2 - 1)
            def _():
                pltpu.async_copy(hw_hbm.at[cur_s.at[b1 + 2]], rows1, sem1)

        if c + 1 < NCHUNK:
            pltpu.make_async_copy(src_hbm.at[wid, pl.ds((c + 1) * CH, CH)],
                                  nxt_s, semi).wait()
            pltpu.make_async_copy(dst_hbm.at[wid, pl.ds((c + 1) * CH, CH)],
                                  nxt_d, semi).wait()

    plsc.subcore_barrier()
    pltpu.sync_copy(acc_sh.at[pl.ds(base, RPW)],
                    out_hbm.at[pl.ds(cid * NPAD + base, RPW)])


def _msg_call(hw, src3, dst3):
    f = pl.kernel(
        _msg_body,
        out_type=jax.ShapeDtypeStruct((NC * NPAD, D), _f32),
        mesh=_MESH,
        scratch_types=[
            pltpu.VMEM((CH, BATCH), jnp.int32),
            pltpu.VMEM((CH, BATCH), jnp.int32),
            pltpu.VMEM((CH, BATCH), jnp.int32),
            pltpu.VMEM((CH, BATCH), jnp.int32),
            pltpu.VMEM((BATCH, D), _f32),
            pltpu.VMEM((BATCH, D), _f32),
            pltpu.VMEM_SHARED((NPAD, D), _f32),
            pltpu.SemaphoreType.DMA,
            pltpu.SemaphoreType.DMA,
            pltpu.SemaphoreType.DMA,
        ],
    )
    return f(hw, src3, dst3)


def _dw_from_coeffs(cpad, ipad):
    r = ipad[:NSPEC_P]
    s = ipad[NSPEC_P:]
    j_a = lax.broadcasted_iota(jnp.int32, (D, NSPEC_P), 0)
    j_b = lax.broadcasted_iota(jnp.int32, (NSPEC_P, D), 1)
    scale = _f32(2.0 * np.pi / D)
    ang_a = ((j_a * r[None, :]) % D).astype(_f32) * scale
    ang_b = ((s[:, None] * j_b) % D).astype(_f32) * scale
    ca = jnp.cos(ang_a) * cpad[None, :]
    sa = jnp.sin(ang_a) * cpad[None, :]
    cb = jnp.cos(ang_b)
    sb = jnp.sin(ang_b)
    dw = (jnp.dot(ca, cb, preferred_element_type=_f32, precision=_HIGH)
          - jnp.dot(sa, sb, preferred_element_type=_f32, precision=_HIGH))
    return dw * _f32(ALPHA / (D * D))


def _fourier_body(x_ref, w1_ref, c1_ref, i1_ref, c2_ref, i2_ref,
                  hw1_ref, embf_ref, dw1_s, dw2_s):
    @pl.when(pl.program_id(0) == 0)
    def _():
        dw1_s[...] = _dw_from_coeffs(c1_ref[0], i1_ref[0])
        dw2_s[...] = _dw_from_coeffs(c2_ref[0], i2_ref[0])

    xb = x_ref[...]
    hw1_ref[...] = jnp.dot(xb, w1_ref[...],
                           preferred_element_type=_f32, precision=_HIGH)
    xf = jnp.maximum(
        jnp.dot(xb, dw1_s[...], preferred_element_type=_f32, precision=_HIGH),
        0.0)
    embf_ref[...] = jnp.dot(xf, dw2_s[...],
                            preferred_element_type=_f32, precision=_HIGH)


def _fourier_call(x, w1, cp1, ip1, cp2, ip2):
    full = lambda shape: pl.BlockSpec(shape, lambda i: (0, 0))
    return pl.pallas_call(
        _fourier_body,
        grid=(GRID,),
        in_specs=[
            pl.BlockSpec((BROW, D), lambda i: (i, 0)),
            full((D, D)),
            full((1, NSPEC_P)),
            full((1, 2 * NSPEC_P)),
            full((1, NSPEC_P)),
            full((1, 2 * NSPEC_P)),
        ],
        out_specs=[
            pl.BlockSpec((BROW, D), lambda i: (i, 0)),
            pl.BlockSpec((BROW, D), lambda i: (i, 0)),
        ],
        out_shape=[
            jax.ShapeDtypeStruct((N, D), _f32),
            jax.ShapeDtypeStruct((N, D), _f32),
        ],
        scratch_shapes=[
            pltpu.VMEM((D, D), _f32),
            pltpu.VMEM((D, D), _f32),
        ],
    )(x, w1, cp1, ip1, cp2, ip2)


def _prep_body(h0_ref, h1_ref, hw1_ref, dinv_ref, hw1s_ref):
    deg = h0_ref[:, 0:1] + h1_ref[:, 0:1] + 1.0
    dinv = lax.rsqrt(deg)
    dinv_b = jnp.broadcast_to(dinv, hw1_ref.shape)
    dinv_ref[...] = dinv_b
    hw1s_ref[...] = hw1_ref[...] * dinv_b


def _prep_call(h0, h1, hw1):
    return pl.pallas_call(
        _prep_body,
        grid=(GRID,),
        in_specs=[
            pl.BlockSpec((BROW, D), lambda i: (i, 0)),
            pl.BlockSpec((BROW, D), lambda i: (i, 0)),
            pl.BlockSpec((BROW, D), lambda i: (i, 0)),
        ],
        out_specs=[
            pl.BlockSpec((BROW, D), lambda i: (i, 0)),
            pl.BlockSpec((BROW, D), lambda i: (i, 0)),
        ],
        out_shape=[
            jax.ShapeDtypeStruct((N, D), _f32),
            jax.ShapeDtypeStruct((N, D), _f32),
        ],
    )(h0, h1, hw1)


def _mid_body(a0_ref, a1_ref, hw1s_ref, dinv_ref, b1_ref, w2_ref, hw2s_ref):
    h1 = jnp.maximum(
        dinv_ref[...] * (a0_ref[...] + a1_ref[...] + hw1s_ref[...])
        + b1_ref[...], 0.0)
    hw2s_ref[...] = dinv_ref[...] * jnp.dot(
        h1, w2_ref[...], preferred_element_type=_f32, precision=_HIGH)


def _mid_call(a0, a1, hw1s, dinv, b1r, w2):
    row = pl.BlockSpec((BROW, D), lambda i: (i, 0))
    return pl.pallas_call(
        _mid_body,
        grid=(GRID,),
        in_specs=[row, row, row, row,
                  pl.BlockSpec((1, D), lambda i: (0, 0)),
                  pl.BlockSpec((D, D), lambda i: (0, 0))],
        out_specs=row,
        out_shape=jax.ShapeDtypeStruct((N, D), _f32),
    )(a0, a1, hw1s, dinv, b1r, w2)


def _final_body(a0_ref, a1_ref, hw2s_ref, dinv_ref, b2_ref, embf_ref,
                base_ref, total_ref):
    base = (dinv_ref[...] * (a0_ref[...] + a1_ref[...] + hw2s_ref[...])
            + b2_ref[...])
    base_ref[...] = base
    total_ref[...] = base + embf_ref[...]


def _final_call(a0, a1, hw2s, dinv, b2r, embf):
    row = pl.BlockSpec((BROW, D), lambda i: (i, 0))
    return pl.pallas_call(
        _final_body,
        grid=(GRID,),
        in_specs=[row, row, row, row,
                  pl.BlockSpec((1, D), lambda i: (0, 0)),
                  row],
        out_specs=[row, row],
        out_shape=[
            jax.ShapeDtypeStruct((N, D), _f32),
            jax.ShapeDtypeStruct((N, D), _f32),
        ],
    )(a0, a1, hw2s, dinv, b2r, embf)


def kernel(x, edge_index, W1, b1, W2, b2, c1, c2, idx1, idx2):
    src = edge_index[0]
    dst = edge_index[1]
    npad = EP - E
    pad_ids = jnp.arange(npad, dtype=jnp.int32)
    psrc = jnp.concatenate([src, pad_ids % 997])
    pdst = jnp.concatenate([dst, N + (pad_ids % 112)])
    src3 = psrc.reshape(NW, KB, BATCH)
    dst3 = pdst.reshape(NW, KB, BATCH)

    def pack(c, idx):
        cp = jnp.zeros((1, NSPEC_P), _f32).at[0, :NSPEC].set(c)
        ip = jnp.zeros((1, 2 * NSPEC_P), jnp.int32)
        ip = ip.at[0, :NSPEC].set(idx[0]).at[0, NSPEC_P:NSPEC_P + NSPEC].set(idx[1])
        return cp, ip

    cp1, ip1 = pack(c1, idx1)
    cp2, ip2 = pack(c2, idx2)

    hist = _deg_call(dst3)
    hw1, embf = _fourier_call(x, W1, cp1, ip1, cp2, ip2)

    h0 = hist[:N]
    h1 = hist[NPAD:NPAD + N]
    dinv, hw1s = _prep_call(h0, h1, hw1)

    def msg(hw):
        acc = _msg_call(hw, src3, dst3)
        return acc[:N], acc[NPAD:NPAD + N]

    a10, a11 = msg(hw1s)
    hw2s = _mid_call(a10, a11, hw1s, dinv, b1.reshape(1, D), W2)

    a20, a21 = msg(hw2s)
    emb_base, emb_total = _final_call(a20, a21, hw2s, dinv,
                                      b2.reshape(1, D), embf)
    return (emb_total, emb_base, embf)
